# baseline (speedup 1.0000x reference)
import jax
import jax.numpy as jnp
from jax.experimental import pallas as pl

_BATCH = 16384
_BITS = 128
_ROWS_PER_BLOCK = 1024

_ROT_A = (13, 15, 26, 6)
_ROT_B = (17, 29, 16, 24)


def _threefry_bits(x1):
    ks = (jnp.uint32(0), jnp.uint32(42), jnp.uint32(0 ^ 42 ^ 0x1BD11BDA))

    def rotl(v, d):
        return (v << jnp.uint32(d)) | (v >> jnp.uint32(32 - d))

    x0 = x1
    x1 = x0 ^ rotl(x1, _ROT_A[0])
    for r in _ROT_A[1:]:
        x0 = x0 + x1
        x1 = rotl(x1, r)
        x1 = x0 ^ x1
    x0 = x0 + ks[1]
    x1 = x1 + (ks[2] + jnp.uint32(1))
    for i in range(1, 5):
        for r in (_ROT_A if i % 2 == 0 else _ROT_B):
            x0 = x0 + x1
            x1 = rotl(x1, r)
            x1 = x0 ^ x1
        x0 = x0 + ks[(i + 1) % 3]
        x1 = x1 + (ks[(i + 2) % 3] + jnp.uint32(i + 1))
    return x0 ^ x1


def _u_from_bits(bits):
    return jax.lax.bitcast_convert_type(
        (bits >> jnp.uint32(9)) | jnp.uint32(0x3F800000), jnp.float32) - 1.0


def _tc_body(x_ref, o_ref):
    p = pl.program_id(0)
    base0 = p * _ROWS_PER_BLOCK * _BITS + 42
    shape = (_BITS, _BITS)
    tile_iota = (
        (jax.lax.broadcasted_iota(jnp.uint32, shape, 1) << jnp.uint32(7))
        + jax.lax.broadcasted_iota(jnp.uint32, shape, 0))
    for k in range(_ROWS_PER_BLOCK // _BITS):
        base = jnp.uint32(base0 + k * _BITS * _BITS) + tile_iota
        u = _u_from_bits(_threefry_bits(base))
        xb = jnp.broadcast_to(x_ref[k:k + 1, :], shape)
        m = jnp.where(u < xb, 1.0, 0.0)
        o_ref[pl.ds(k * _BITS, _BITS), :] = m.T


def kernel(x):
    x2 = x.reshape(_BATCH // _BITS, _BITS)
    return pl.pallas_call(
        _tc_body,
        grid=(_BATCH // _ROWS_PER_BLOCK,),
        in_specs=[pl.BlockSpec(
            (_ROWS_PER_BLOCK // _BITS, _BITS), lambda p: (p, 0))],
        out_specs=pl.BlockSpec((_ROWS_PER_BLOCK, _BITS), lambda p: (p, 0)),
        out_shape=jax.ShapeDtypeStruct((_BATCH, _BITS), jnp.float32),
    )(x2)
